# Initial kernel scaffold; baseline (speedup 1.0000x reference)
import jax
import jax.numpy as jnp
from jax.experimental import pallas as pl


def kernel(p, n, x, o, W, gamma, beta):
    raise NotImplementedError("write your pallas kernel here")


# speedup vs baseline: 8.6663x; 8.6663x over previous
import functools

import jax
import jax.numpy as jnp
from jax import lax
from jax.experimental import pallas as pl
from jax.experimental.pallas import tpu as pltpu
from jax.experimental.pallas import tpu_sc as plsc

B = 4
NP = 8192
STRIDE = 4
NSAMPLE = 16
CIN = 32
COUT = 64
EPS = 1e-5
M_PER = NP // STRIDE
M = B * M_PER
NROWS = B * NP
DT = 128
SUB = 8
LANES = NP // SUB
BIG_I = 2 ** 30


def _fps_body(pxyz_ref, idx_ref):
    b = pl.program_id(0)
    px = pxyz_ref[0]
    py = pxyz_ref[1]
    pz = pxyz_ref[2]
    row = lax.broadcasted_iota(jnp.int32, (SUB, LANES), 0)
    col = lax.broadcasted_iota(jnp.int32, (SUB, LANES), 1)
    iota = row * LANES + col
    base = b * NP
    idx_ref[pl.ds(0, 1), :] = jnp.full((1, 1), base, jnp.int32)

    def body(i, carry):
        dists, qx, qy, qz = carry
        dx = px - qx
        dy = py - qy
        dz = pz - qz
        d = (dx * dx + dz * dz) + dy * dy
        dists = jnp.minimum(dists, d)
        mx = jnp.max(dists, keepdims=True)
        cand = jnp.where(dists == mx, iota, jnp.int32(BIG_I))
        nxt = jnp.min(cand, keepdims=True)
        oh = iota == nxt
        nqx = jnp.sum(jnp.where(oh, px, 0.0), keepdims=True)
        nqy = jnp.sum(jnp.where(oh, py, 0.0), keepdims=True)
        nqz = jnp.sum(jnp.where(oh, pz, 0.0), keepdims=True)
        idx_ref[pl.ds(i, 1), :] = nxt + base
        return dists, nqx, nqy, nqz

    dists0 = jnp.full((SUB, LANES), 1e10, jnp.float32)
    lax.fori_loop(1, M_PER, body,
                  (dists0, px[0:1, 0:1], py[0:1, 0:1], pz[0:1, 0:1]))


def _fps(pxyz):
    return pl.pallas_call(
        _fps_body,
        grid=(B,),
        in_specs=[pl.BlockSpec((3, SUB, LANES), lambda b: (b, 0, 0))],
        out_specs=pl.BlockSpec((M_PER, 1), lambda b: (b, 0)),
        out_shape=jax.ShapeDtypeStruct((M, 1), jnp.int32),
    )(pxyz)


BQ = 256


def _knn_body(q_ref, kT_ref, nbr_ref):
    b = pl.program_id(0)
    q = q_ref[...]
    kT = kT_ref[0]
    cross = lax.dot_general(q, kT, (((1,), (0,)), ((), ())),
                            precision=lax.Precision.DEFAULT,
                            preferred_element_type=jnp.float32)
    qx = q[:, 0:1]
    qy = q[:, 1:2]
    qz = q[:, 2:3]
    dq2 = (qx * qx + qz * qz) + qy * qy
    kx = kT[0:1, :]
    ky = kT[1:2, :]
    kz = kT[2:3, :]
    dk2 = (kx * kx + kz * kz) + ky * ky
    d = (dq2 - 2.0 * cross) + dk2
    iota = lax.broadcasted_iota(jnp.int32, (BQ, NP), 1)
    base = b * NP
    for s in range(NSAMPLE):
        mn = jnp.min(d, axis=1, keepdims=True)
        cand = jnp.where(d == mn, iota, jnp.int32(BIG_I))
        ids = jnp.min(cand, axis=1, keepdims=True)
        nbr_ref[:, pl.ds(s, 1)] = ids + base
        d = jnp.where(iota == ids, jnp.float32(jnp.inf), d)


def _knn(n_p, kTall):
    nb = M_PER // BQ
    return pl.pallas_call(
        _knn_body,
        grid=(B, nb),
        in_specs=[
            pl.BlockSpec((BQ, 3), lambda b, j: (b * nb + j, 0)),
            pl.BlockSpec((1, 3, NP), lambda b, j: (b, 0, 0)),
        ],
        out_specs=pl.BlockSpec((BQ, NSAMPLE), lambda b, j: (b * nb + j, 0)),
        out_shape=jax.ShapeDtypeStruct((M, NSAMPLE), jnp.int32),
    )(n_p, kTall)


def _make_sc_gather(nrows_out, chunk):
    info = plsc.get_sparse_core_info()
    nc, ns = info.num_cores, info.num_subcores
    nw = nc * ns
    b_per_w = nrows_out // nw
    nchunks = b_per_w // chunk
    mesh = plsc.VectorSubcoreMesh(core_axis_name="c", subcore_axis_name="s")

    @functools.partial(
        pl.kernel, mesh=mesh,
        out_type=jax.ShapeDtypeStruct((nrows_out, DT), jnp.float32),
        scratch_types=[
            pltpu.VMEM((chunk,), jnp.int32),
            pltpu.VMEM((chunk, DT), jnp.float32),
            pltpu.SemaphoreType.DMA,
        ],
    )
    def gk(table_hbm, idx_hbm, out_hbm, idx_v, rows_v, sem):
        wid = lax.axis_index("s") * nc + lax.axis_index("c")
        base = wid * b_per_w
        for j in range(nchunks):
            off = base + j * chunk
            pltpu.sync_copy(idx_hbm.at[pl.ds(off, chunk)], idx_v)
            pltpu.async_copy(table_hbm.at[idx_v], rows_v, sem).wait()
            pltpu.sync_copy(rows_v, out_hbm.at[pl.ds(off, chunk)])

    return gk


BM = 128
BR = BM * NSAMPLE


def _mlp_body(g_ref, q_ref, w_ref, tmax_ref, tmin_ref, sums_ref, acc_ref):
    i = pl.program_id(0)
    g = g_ref[...]
    w = w_ref[...]
    t = lax.dot_general(g, w, (((1,), (0,)), ((), ())),
                        precision=lax.Precision.DEFAULT,
                        preferred_element_type=jnp.float32)
    q = q_ref[...]
    w3 = w_ref[0:3, :]
    c = lax.dot_general(q, w3, (((1,), (0,)), ((), ())),
                        precision=lax.Precision.DEFAULT,
                        preferred_element_type=jnp.float32)
    t3 = t.reshape(BM, NSAMPLE, COUT)
    tc = t3 - c[:, None, :]
    tmax_ref[...] = jnp.max(tc, axis=1)
    tmin_ref[...] = jnp.min(tc, axis=1)
    s1 = jnp.sum(jnp.sum(tc, axis=1), axis=0, keepdims=True)
    s2 = jnp.sum(jnp.sum(tc * tc, axis=1), axis=0, keepdims=True)

    @pl.when(i == 0)
    def _():
        acc_ref[...] = jnp.zeros((SUB, COUT), jnp.float32)

    acc_ref[0:1, :] += s1
    acc_ref[1:2, :] += s2
    sums_ref[...] = acc_ref[...]


def _mlp(g2, n_p, wpad):
    ng = M // BM
    return pl.pallas_call(
        _mlp_body,
        grid=(ng,),
        in_specs=[
            pl.BlockSpec((BR, DT), lambda i: (i, 0)),
            pl.BlockSpec((BM, 3), lambda i: (i, 0)),
            pl.BlockSpec((DT, COUT), lambda i: (0, 0)),
        ],
        out_specs=[
            pl.BlockSpec((BM, COUT), lambda i: (i, 0)),
            pl.BlockSpec((BM, COUT), lambda i: (i, 0)),
            pl.BlockSpec((SUB, COUT), lambda i: (0, 0)),
        ],
        out_shape=[
            jax.ShapeDtypeStruct((M, COUT), jnp.float32),
            jax.ShapeDtypeStruct((M, COUT), jnp.float32),
            jax.ShapeDtypeStruct((SUB, COUT), jnp.float32),
        ],
        scratch_shapes=[pltpu.VMEM((SUB, COUT), jnp.float32)],
    )(g2, n_p, wpad)


def _bn_body(sums_ref, tmax_ref, tmin_ref, gamma_ref, beta_ref, out_ref):
    s1 = sums_ref[0:1, :]
    s2 = sums_ref[1:2, :]
    cnt = jnp.float32(M * NSAMPLE)
    mean = s1 / cnt
    var = s2 / cnt - mean * mean
    a = gamma_ref[...] / jnp.sqrt(var + EPS)
    bb = beta_ref[...] - mean * a
    sel = jnp.where(a >= 0.0, tmax_ref[...], tmin_ref[...])
    out_ref[...] = jnp.maximum(sel * a + bb, 0.0)


def _bn(sums, tmax, tmin, gamma, beta):
    return pl.pallas_call(
        _bn_body,
        out_shape=jax.ShapeDtypeStruct((M, COUT), jnp.float32),
    )(sums, tmax, tmin, gamma.reshape(1, COUT), beta.reshape(1, COUT))


def kernel(p, n, x, o, W, gamma, beta):
    pb = p.reshape(B, NP, 3)
    pT = pb.transpose(0, 2, 1)
    pxyz = pT.reshape(B * 3, SUB, LANES)
    kTall = pT
    table = jnp.concatenate(
        [p, x, n, jnp.zeros((NROWS, DT - 3 - CIN - 3), p.dtype)], axis=1)
    wpad = jnp.concatenate(
        [W, jnp.zeros((DT - 3 - CIN, COUT), W.dtype)], axis=0)

    idx = _fps(pxyz)
    g1 = _make_sc_gather(M, 128)(table, idx.reshape(M))
    n_p = g1[:, 0:3]
    n_n = g1[:, 3 + CIN:3 + CIN + 3]

    nbr = _knn(n_p, kTall)
    g2 = _make_sc_gather(M * NSAMPLE, 128)(table, nbr.reshape(M * NSAMPLE))

    tmax, tmin, sums = _mlp(g2, n_p, wpad)
    x_out = _bn(sums, tmax, tmin, gamma, beta)

    n_o = jnp.arange(1, B + 1, dtype=jnp.int32) * M_PER
    return n_p, n_n, x_out, n_o


# speedup vs baseline: 20.1068x; 2.3201x over previous
import functools

import jax
import jax.numpy as jnp
from jax import lax
from jax.experimental import pallas as pl
from jax.experimental.pallas import tpu as pltpu
from jax.experimental.pallas import tpu_sc as plsc

B = 4
NP = 8192
STRIDE = 4
NSAMPLE = 16
CIN = 32
COUT = 64
EPS = 1e-5
M_PER = NP // STRIDE
M = B * M_PER
NROWS = B * NP
DT = 128
SUB = 8
LANES = NP // SUB
BIG_I = 2 ** 30


def _fps_body(pxyz_ref, idx_ref):
    px = pxyz_ref[0]
    py = pxyz_ref[1]
    pz = pxyz_ref[2]
    row = lax.broadcasted_iota(jnp.int32, (B, SUB, LANES), 1)
    col = lax.broadcasted_iota(jnp.int32, (B, SUB, LANES), 2)
    iota = row * LANES + col
    base_v = lax.broadcasted_iota(jnp.int32, (B, 1, 1), 0) * NP

    def _rmax(v):
        return jnp.max(jnp.max(v, axis=2, keepdims=True), axis=1, keepdims=True)

    def _rmin(v):
        return jnp.min(jnp.min(v, axis=2, keepdims=True), axis=1, keepdims=True)

    def _rsum(v):
        return jnp.sum(jnp.sum(v, axis=2, keepdims=True), axis=1, keepdims=True)

    orow = lax.broadcasted_iota(jnp.int32, (B, M_PER // LANES, LANES), 1)
    ocol = lax.broadcasted_iota(jnp.int32, (B, M_PER // LANES, LANES), 2)
    oiota = orow * LANES + ocol
    acc0 = jnp.where(oiota == 0, base_v, 0)

    def body(i, carry):
        dists, acc, qx, qy, qz = carry
        dx = px - qx
        dy = py - qy
        dz = pz - qz
        d = (dx * dx + dz * dz) + dy * dy
        dists = jnp.minimum(dists, d)
        mx = _rmax(dists)
        cand = jnp.where(dists == mx, iota, jnp.int32(BIG_I))
        nxt = _rmin(cand)
        oh = iota == nxt
        nqx = _rsum(jnp.where(oh, px, 0.0))
        nqy = _rsum(jnp.where(oh, py, 0.0))
        nqz = _rsum(jnp.where(oh, pz, 0.0))
        acc = jnp.where(oiota == i, nxt + base_v, acc)
        return dists, acc, nqx, nqy, nqz

    dists0 = jnp.full((B, SUB, LANES), 1e10, jnp.float32)
    _, acc, _, _, _ = lax.fori_loop(
        1, M_PER, body,
        (dists0, acc0, px[:, 0:1, 0:1], py[:, 0:1, 0:1], pz[:, 0:1, 0:1]))
    idx_ref[...] = acc


def _fps(pxyz):
    return pl.pallas_call(
        _fps_body,
        out_shape=jax.ShapeDtypeStruct((B, M_PER // LANES, LANES), jnp.int32),
    )(pxyz)


BQ = 256


def _knn_body(q_ref, kT_ref, nbr_ref):
    b = pl.program_id(0)
    q = q_ref[...]
    kT = kT_ref[0]
    cross = lax.dot_general(q, kT, (((1,), (0,)), ((), ())),
                            precision=lax.Precision.DEFAULT,
                            preferred_element_type=jnp.float32)
    qx = q[:, 0:1]
    qy = q[:, 1:2]
    qz = q[:, 2:3]
    dq2 = (qx * qx + qz * qz) + qy * qy
    kx = kT[0:1, :]
    ky = kT[1:2, :]
    kz = kT[2:3, :]
    dk2 = (kx * kx + kz * kz) + ky * ky
    d = (dq2 - 2.0 * cross) + dk2
    iota = lax.broadcasted_iota(jnp.int32, (BQ, NP), 1)
    base = b * NP
    for s in range(NSAMPLE):
        mn = jnp.min(d, axis=1, keepdims=True)
        cand = jnp.where(d == mn, iota, jnp.int32(BIG_I))
        ids = jnp.min(cand, axis=1, keepdims=True)
        nbr_ref[:, pl.ds(s, 1)] = ids + base
        d = jnp.where(iota == ids, jnp.float32(jnp.inf), d)


def _knn(n_p, kTall):
    nb = M_PER // BQ
    return pl.pallas_call(
        _knn_body,
        grid=(B, nb),
        in_specs=[
            pl.BlockSpec((BQ, 3), lambda b, j: (b * nb + j, 0)),
            pl.BlockSpec((1, 3, NP), lambda b, j: (b, 0, 0)),
        ],
        out_specs=pl.BlockSpec((BQ, NSAMPLE), lambda b, j: (b * nb + j, 0)),
        out_shape=jax.ShapeDtypeStruct((M, NSAMPLE), jnp.int32),
    )(n_p, kTall)


def _make_sc_gather(nrows_out, chunk):
    info = plsc.get_sparse_core_info()
    nc, ns = info.num_cores, info.num_subcores
    nw = nc * ns
    b_per_w = nrows_out // nw
    nchunks = b_per_w // chunk
    mesh = plsc.VectorSubcoreMesh(core_axis_name="c", subcore_axis_name="s")

    @functools.partial(
        pl.kernel, mesh=mesh,
        out_type=jax.ShapeDtypeStruct((nrows_out, DT), jnp.float32),
        scratch_types=[
            pltpu.VMEM((chunk,), jnp.int32),
            pltpu.VMEM((chunk, DT), jnp.float32),
            pltpu.SemaphoreType.DMA,
        ],
    )
    def gk(table_hbm, idx_hbm, out_hbm, idx_v, rows_v, sem):
        wid = lax.axis_index("s") * nc + lax.axis_index("c")
        base = wid * b_per_w
        for j in range(nchunks):
            off = base + j * chunk
            pltpu.sync_copy(idx_hbm.at[pl.ds(off, chunk)], idx_v)
            pltpu.async_copy(table_hbm.at[idx_v], rows_v, sem).wait()
            pltpu.sync_copy(rows_v, out_hbm.at[pl.ds(off, chunk)])

    return gk


BM = 128
BR = BM * NSAMPLE


def _mlp_body(g_ref, q_ref, w_ref, tmax_ref, tmin_ref, sums_ref, acc_ref):
    i = pl.program_id(0)
    g = g_ref[...]
    w = w_ref[...]
    t = lax.dot_general(g, w, (((1,), (0,)), ((), ())),
                        precision=lax.Precision.DEFAULT,
                        preferred_element_type=jnp.float32)
    q = q_ref[...]
    w3 = w_ref[0:3, :]
    c = lax.dot_general(q, w3, (((1,), (0,)), ((), ())),
                        precision=lax.Precision.DEFAULT,
                        preferred_element_type=jnp.float32)
    t3 = t.reshape(BM, NSAMPLE, COUT)
    tc = t3 - c[:, None, :]
    tmax_ref[...] = jnp.max(tc, axis=1)
    tmin_ref[...] = jnp.min(tc, axis=1)
    s1 = jnp.sum(jnp.sum(tc, axis=1), axis=0, keepdims=True)
    s2 = jnp.sum(jnp.sum(tc * tc, axis=1), axis=0, keepdims=True)

    @pl.when(i == 0)
    def _():
        acc_ref[...] = jnp.zeros((SUB, COUT), jnp.float32)

    acc_ref[0:1, :] += s1
    acc_ref[1:2, :] += s2
    sums_ref[...] = acc_ref[...]


def _mlp(g2, n_p, wpad):
    ng = M // BM
    return pl.pallas_call(
        _mlp_body,
        grid=(ng,),
        in_specs=[
            pl.BlockSpec((BR, DT), lambda i: (i, 0)),
            pl.BlockSpec((BM, 3), lambda i: (i, 0)),
            pl.BlockSpec((DT, COUT), lambda i: (0, 0)),
        ],
        out_specs=[
            pl.BlockSpec((BM, COUT), lambda i: (i, 0)),
            pl.BlockSpec((BM, COUT), lambda i: (i, 0)),
            pl.BlockSpec((SUB, COUT), lambda i: (0, 0)),
        ],
        out_shape=[
            jax.ShapeDtypeStruct((M, COUT), jnp.float32),
            jax.ShapeDtypeStruct((M, COUT), jnp.float32),
            jax.ShapeDtypeStruct((SUB, COUT), jnp.float32),
        ],
        scratch_shapes=[pltpu.VMEM((SUB, COUT), jnp.float32)],
    )(g2, n_p, wpad)


def _bn_body(sums_ref, tmax_ref, tmin_ref, gamma_ref, beta_ref, out_ref):
    s1 = sums_ref[0:1, :]
    s2 = sums_ref[1:2, :]
    cnt = jnp.float32(M * NSAMPLE)
    mean = s1 / cnt
    var = s2 / cnt - mean * mean
    a = gamma_ref[...] / jnp.sqrt(var + EPS)
    bb = beta_ref[...] - mean * a
    sel = jnp.where(a >= 0.0, tmax_ref[...], tmin_ref[...])
    out_ref[...] = jnp.maximum(sel * a + bb, 0.0)


def _bn(sums, tmax, tmin, gamma, beta):
    return pl.pallas_call(
        _bn_body,
        out_shape=jax.ShapeDtypeStruct((M, COUT), jnp.float32),
    )(sums, tmax, tmin, gamma.reshape(1, COUT), beta.reshape(1, COUT))


def kernel(p, n, x, o, W, gamma, beta):
    pb = p.reshape(B, NP, 3)
    pT = pb.transpose(0, 2, 1)
    pxyz = pT.transpose(1, 0, 2).reshape(3, B, SUB, LANES)
    kTall = pT
    table = jnp.concatenate(
        [p, x, n, jnp.zeros((NROWS, DT - 3 - CIN - 3), p.dtype)], axis=1)
    wpad = jnp.concatenate(
        [W, jnp.zeros((DT - 3 - CIN, COUT), W.dtype)], axis=0)

    idx = _fps(pxyz)
    g1 = _make_sc_gather(M, 128)(table, idx.reshape(M))
    n_p = g1[:, 0:3]
    n_n = g1[:, 3 + CIN:3 + CIN + 3]

    nbr = _knn(n_p, kTall)
    g2 = _make_sc_gather(M * NSAMPLE, 128)(table, nbr.reshape(M * NSAMPLE))

    tmax, tmin, sums = _mlp(g2, n_p, wpad)
    x_out = _bn(sums, tmax, tmin, gamma, beta)

    n_o = jnp.arange(1, B + 1, dtype=jnp.int32) * M_PER
    return n_p, n_n, x_out, n_o
